# baseline (speedup 1.0000x reference)
import functools

import jax
import jax.numpy as jnp
from jax.experimental import pallas as pl
from jax.experimental.pallas import tpu as pltpu


_TAP2 = {-1: (1, -1), 0: (0, 0), 1: (1, 0)}

_RS = 8
_IS = 56


def _shift_to_patch(patch, read, s, c0, c1, mask=None):
    Mt = patch.shape[0]
    if s > 0:
        v = read(s, Mt)
        if mask is not None:
            v = jnp.where(mask[0:Mt - s], v, jnp.zeros_like(v))
            patch[Mt - s:Mt, c0:c1] = jnp.zeros((s, c1 - c0), patch.dtype)
        patch[0:Mt - s, c0:c1] = v
    elif s == 0:
        patch[:, c0:c1] = read(0, Mt)
    else:
        v = read(0, Mt + s)
        if mask is not None:
            v = jnp.where(mask[-s:Mt], v, jnp.zeros_like(v))
        patch[0:-s, c0:c1] = jnp.zeros((-s, c1 - c0), patch.dtype)
        patch[-s:Mt, c0:c1] = v


def _fe_kernel(x3_ref, sel_ref,
               w11_ref, b11_ref, wds_ref, bds_ref, w12_ref, b12_ref,
               w21_ref, b21_ref, w22_ref, b22_ref, wfc_ref, bfc_ref,
               o_ref,
               xp, patch, hA, y1f, idt_scr,
               *, G, Ho, Wo, Cin, N):
    Mt = G * _IS
    r = jax.lax.broadcasted_iota(jnp.int32, (Mt, 1), 0)
    rr = r % _IS
    valid = ((r % _RS) < Wo) & (rr < (Ho - 1) * _RS + Wo)
    i_of_r = rr // _RS
    m_top = i_of_r != 0
    m_bot = i_of_r != (Ho - 1)

    GB = 4 if G % 4 == 0 else 1
    for g in range(0, G, GB):
        xg = x3_ref[g:g + GB].reshape(GB * Cin, -1).astype(jnp.bfloat16)
        yg = jax.lax.dot_general(
            sel_ref[...], xg,
            (((1,), (1,)), ((), ())),
            preferred_element_type=jnp.float32).astype(jnp.bfloat16)
        for k in range(GB):
            for ph in range(4):
                xp[ph, (g + k) * _IS:(g + k + 1) * _IS, :] = \
                    yg[ph * _IS:(ph + 1) * _IS, k * Cin:(k + 1) * Cin]

    for ky in range(3):
        p, dy = _TAP2[ky - 1]
        for kx in range(3):
            q, dx = _TAP2[kx - 1]
            t = ky * 3 + kx
            _shift_to_patch(patch, lambda a, b: xp[2 * p + q, a:b, :],
                            dy * _RS + dx, t * Cin, (t + 1) * Cin,
                            mask=m_top if dy != 0 else None)
    acc = jnp.dot(patch[:, :9 * Cin], w11_ref[...],
                  preferred_element_type=jnp.float32) + b11_ref[...]
    hA[...] = jnp.where(valid, jnp.maximum(acc, 0.0), 0.0).astype(hA.dtype)

    idt_scr[...] = jnp.dot(xp[0], wds_ref[...],
                           preferred_element_type=jnp.float32) + bds_ref[...]

    for ky in range(3):
        for kx in range(3):
            t = ky * 3 + kx
            _shift_to_patch(patch, lambda a, b: hA[a:b, :],
                            (ky - 1) * _RS + (kx - 1), t * N, (t + 1) * N,
                            mask=(m_top, None, m_bot)[ky])
    acc = jnp.dot(patch[...], w12_ref[...],
                  preferred_element_type=jnp.float32) + b12_ref[...]
    y1f[...] = jnp.where(valid, jnp.maximum(acc + idt_scr[...], 0.0),
                         0.0).astype(y1f.dtype)

    for ky in range(3):
        for kx in range(3):
            t = ky * 3 + kx
            _shift_to_patch(patch, lambda a, b: y1f[a:b, :],
                            (ky - 1) * _RS + (kx - 1), t * N, (t + 1) * N,
                            mask=(m_top, None, m_bot)[ky])
    acc = jnp.dot(patch[...], w21_ref[...],
                  preferred_element_type=jnp.float32) + b21_ref[...]
    hA[...] = jnp.where(valid, jnp.maximum(acc, 0.0), 0.0).astype(hA.dtype)

    for ky in range(3):
        for kx in range(3):
            t = ky * 3 + kx
            _shift_to_patch(patch, lambda a, b: hA[a:b, :],
                            (ky - 1) * _RS + (kx - 1), t * N, (t + 1) * N,
                            mask=(m_top, None, m_bot)[ky])
    acc = jnp.dot(patch[...], w22_ref[...],
                  preferred_element_type=jnp.float32) + b22_ref[...]
    y2 = jnp.where(valid, jnp.maximum(acc + y1f[...].astype(jnp.float32), 0.0),
                   0.0)

    g_id = jax.lax.broadcasted_iota(jnp.int32, (G, Mt), 0)
    c_id = jax.lax.broadcasted_iota(jnp.int32, (G, Mt), 1)
    sel = ((c_id // _IS == g_id) & ((c_id % _RS) < Wo)
           & ((c_id % _IS) < (Ho - 1) * _RS + Wo))
    pmat = jnp.where(sel, jnp.float32(1.0 / (Ho * Wo)), jnp.float32(0.0))
    pooled = jnp.dot(pmat, y2, preferred_element_type=jnp.float32)

    h = jnp.dot(pooled, wfc_ref[...], preferred_element_type=jnp.float32)
    h = jnp.maximum(h + bfc_ref[...], 0.0)
    l2 = jnp.sqrt(jnp.sum(h * h, axis=1, keepdims=True))
    denom = jnp.sqrt(jnp.maximum(l2, 1e-12)) * jnp.float32(2.0 ** 0.5)
    o_ref[...] = h / denom


@jax.jit
def _fe_forward(x_nchw, params):
    B, C, H, W = x_nchw.shape
    Hh, Wh = H // 2, W // 2
    N = params["w11"].shape[1]
    G = 8 if B % 8 == 0 else B
    Mt = G * _IS
    S = H * W

    x3 = x_nchw.reshape(B, C, S)

    r_id = jax.lax.broadcasted_iota(jnp.int32, (4 * _IS, S), 0)
    s_id = jax.lax.broadcasted_iota(jnp.int32, (4 * _IS, S), 1)
    ph, rr = r_id // _IS, r_id % _IS
    i, j = rr // _RS, rr % _RS
    sigma = (2 * i + ph // 2) * W + 2 * j + ph % 2
    ok = (i < Hh) & (j < Wh)
    sel2 = ((s_id == sigma) & ok).astype(jnp.bfloat16)

    flops = B * (2 * Hh * Wh * 9 * C * N + 2 * Hh * Wh * C * N
                 + 3 * 2 * Hh * Wh * 9 * N * N + 2 * N * N)
    weight_keys = ("w11", "b11", "wds", "bds", "w12", "b12", "w21", "b21",
                   "w22", "b22", "wfc", "bfc")
    w_bytes = sum(int(params[k].size) * params[k].dtype.itemsize
                  for k in weight_keys)
    bytes_accessed = int(x3.size) * 4 + w_bytes + B * N * 4

    resident = pl.BlockSpec(memory_space=pltpu.MemorySpace.VMEM)

    out = pl.pallas_call(
        functools.partial(_fe_kernel, G=G, Ho=Hh, Wo=Wh, Cin=C, N=N),
        out_shape=jax.ShapeDtypeStruct((B, N), jnp.float32),
        grid=(B // G,),
        in_specs=[pl.BlockSpec((G, C, S), lambda b: (b, 0, 0))]
        + [resident] * 13,
        out_specs=pl.BlockSpec((G, N), lambda b: (b, 0)),
        scratch_shapes=[
            pltpu.VMEM((4, Mt, C), jnp.bfloat16),
            pltpu.VMEM((Mt, 9 * N), jnp.bfloat16),
            pltpu.VMEM((Mt, N), jnp.bfloat16),
            pltpu.VMEM((Mt, N), jnp.bfloat16),
            pltpu.VMEM((Mt, N), jnp.float32),
        ],
        compiler_params=pltpu.CompilerParams(
            dimension_semantics=("parallel",),
            vmem_limit_bytes=56 * 1024 * 1024),
        cost_estimate=pl.CostEstimate(flops=flops, transcendentals=4 * B,
                                      bytes_accessed=bytes_accessed),
    )(x3, sel2,
      params["w11"], params["b11"], params["wds"], params["bds"],
      params["w12"], params["b12"], params["w21"], params["b21"],
      params["w22"], params["b22"], params["wfc"], params["bfc"])
    return out


def kernel(x, w11, b11, wds, bds, w12, b12, w21, b21, w22, b22, wfc, bfc):
    params = {"w11": w11, "b11": b11, "wds": wds, "bds": bds,
              "w12": w12, "b12": b12, "w21": w21, "b21": b21,
              "w22": w22, "b22": b22, "wfc": wfc, "bfc": bfc}
    return _fe_forward(x, params)


# speedup vs baseline: 1.0156x; 1.0156x over previous
import functools

import jax
import jax.numpy as jnp
from jax.experimental import pallas as pl
from jax.experimental.pallas import tpu as pltpu


_TAP2 = {-1: (1, -1), 0: (0, 0), 1: (1, 0)}

_RS = 8
_IS = 56


def _shift_to_patch(patch, read, s, c0, c1, mask=None):
    Mt = patch.shape[0]
    if s > 0:
        v = read(s, Mt)
        if mask is not None:
            v = jnp.where(mask[0:Mt - s], v, jnp.zeros_like(v))
            patch[Mt - s:Mt, c0:c1] = jnp.zeros((s, c1 - c0), patch.dtype)
        patch[0:Mt - s, c0:c1] = v
    elif s == 0:
        patch[:, c0:c1] = read(0, Mt)
    else:
        v = read(0, Mt + s)
        if mask is not None:
            v = jnp.where(mask[-s:Mt], v, jnp.zeros_like(v))
        patch[0:-s, c0:c1] = jnp.zeros((-s, c1 - c0), patch.dtype)
        patch[-s:Mt, c0:c1] = v


def _fe_kernel(x3_ref, sel_ref, pmat_ref,
               w11_ref, b11_ref, wds_ref, bds_ref, w12_ref, b12_ref,
               w21_ref, b21_ref, w22_ref, b22_ref, wfc_ref, bfc_ref,
               o_ref,
               xp, patch, hA, y1f, idt_scr,
               *, G, Ho, Wo, Cin, N):
    Mt = G * _IS
    r = jax.lax.broadcasted_iota(jnp.int32, (Mt, 1), 0)
    rr = r % _IS
    valid = ((r % _RS) < Wo) & (rr < (Ho - 1) * _RS + Wo)
    i_of_r = rr // _RS
    m_top = i_of_r != 0
    m_bot = i_of_r != (Ho - 1)

    GB = 4 if G % 4 == 0 else 1
    for g in range(0, G, GB):
        xg = x3_ref[g:g + GB].reshape(GB * Cin, -1).astype(jnp.bfloat16)
        yg = jax.lax.dot_general(
            sel_ref[...], xg,
            (((1,), (1,)), ((), ())),
            preferred_element_type=jnp.float32).astype(jnp.bfloat16)
        for k in range(GB):
            for ph in range(4):
                xp[ph, (g + k) * _IS:(g + k + 1) * _IS, :] = \
                    yg[ph * _IS:(ph + 1) * _IS, k * Cin:(k + 1) * Cin]

    for ky in range(3):
        p, dy = _TAP2[ky - 1]
        for kx in range(3):
            q, dx = _TAP2[kx - 1]
            t = ky * 3 + kx
            _shift_to_patch(patch, lambda a, b: xp[2 * p + q, a:b, :],
                            dy * _RS + dx, t * Cin, (t + 1) * Cin,
                            mask=m_top if dy != 0 else None)
    acc = jnp.dot(patch[:, :9 * Cin], w11_ref[...],
                  preferred_element_type=jnp.float32) + b11_ref[...]
    hA[...] = jnp.where(valid, jnp.maximum(acc, 0.0), 0.0).astype(hA.dtype)

    idt_scr[...] = jnp.dot(xp[0], wds_ref[...],
                           preferred_element_type=jnp.float32) + bds_ref[...]

    for ky in range(3):
        for kx in range(3):
            t = ky * 3 + kx
            _shift_to_patch(patch, lambda a, b: hA[a:b, :],
                            (ky - 1) * _RS + (kx - 1), t * N, (t + 1) * N,
                            mask=(m_top, None, m_bot)[ky])
    acc = jnp.dot(patch[...], w12_ref[...],
                  preferred_element_type=jnp.float32) + b12_ref[...]
    y1f[...] = jnp.where(valid, jnp.maximum(acc + idt_scr[...], 0.0),
                         0.0).astype(y1f.dtype)

    for ky in range(3):
        for kx in range(3):
            t = ky * 3 + kx
            _shift_to_patch(patch, lambda a, b: y1f[a:b, :],
                            (ky - 1) * _RS + (kx - 1), t * N, (t + 1) * N,
                            mask=(m_top, None, m_bot)[ky])
    acc = jnp.dot(patch[...], w21_ref[...],
                  preferred_element_type=jnp.float32) + b21_ref[...]
    hA[...] = jnp.where(valid, jnp.maximum(acc, 0.0), 0.0).astype(hA.dtype)

    for ky in range(3):
        for kx in range(3):
            t = ky * 3 + kx
            _shift_to_patch(patch, lambda a, b: hA[a:b, :],
                            (ky - 1) * _RS + (kx - 1), t * N, (t + 1) * N,
                            mask=(m_top, None, m_bot)[ky])
    acc = jnp.dot(patch[...], w22_ref[...],
                  preferred_element_type=jnp.float32) + b22_ref[...]
    y2 = jnp.where(valid, jnp.maximum(acc + y1f[...].astype(jnp.float32), 0.0),
                   0.0)

    pooled = jnp.dot(pmat_ref[...], y2,
                     preferred_element_type=jnp.float32)

    h = jnp.dot(pooled, wfc_ref[...], preferred_element_type=jnp.float32)
    h = jnp.maximum(h + bfc_ref[...], 0.0)
    l2 = jnp.sqrt(jnp.sum(h * h, axis=1, keepdims=True))
    denom = jnp.sqrt(jnp.maximum(l2, 1e-12)) * jnp.float32(2.0 ** 0.5)
    o_ref[...] = h / denom


@jax.jit
def _fe_forward(x_nchw, params):
    B, C, H, W = x_nchw.shape
    Hh, Wh = H // 2, W // 2
    N = params["w11"].shape[1]
    G = 16 if B % 16 == 0 else B
    Mt = G * _IS
    S = H * W

    x3 = x_nchw.reshape(B, C, S)

    r_id = jax.lax.broadcasted_iota(jnp.int32, (4 * _IS, S), 0)
    s_id = jax.lax.broadcasted_iota(jnp.int32, (4 * _IS, S), 1)
    ph, rr = r_id // _IS, r_id % _IS
    i, j = rr // _RS, rr % _RS
    sigma = (2 * i + ph // 2) * W + 2 * j + ph % 2
    ok = (i < Hh) & (j < Wh)
    sel2 = ((s_id == sigma) & ok).astype(jnp.bfloat16)

    g_id = jax.lax.broadcasted_iota(jnp.int32, (G, G * _IS), 0)
    c_id = jax.lax.broadcasted_iota(jnp.int32, (G, G * _IS), 1)
    psel = ((c_id // _IS == g_id) & ((c_id % _RS) < Wh)
            & ((c_id % _IS) < (Hh - 1) * _RS + Wh))
    pmat = jnp.where(psel, jnp.float32(1.0 / (Hh * Wh)), jnp.float32(0.0))

    flops = B * (2 * Hh * Wh * 9 * C * N + 2 * Hh * Wh * C * N
                 + 3 * 2 * Hh * Wh * 9 * N * N + 2 * N * N)
    weight_keys = ("w11", "b11", "wds", "bds", "w12", "b12", "w21", "b21",
                   "w22", "b22", "wfc", "bfc")
    w_bytes = sum(int(params[k].size) * params[k].dtype.itemsize
                  for k in weight_keys)
    bytes_accessed = int(x3.size) * 4 + w_bytes + B * N * 4

    resident = pl.BlockSpec(memory_space=pltpu.MemorySpace.VMEM)

    out = pl.pallas_call(
        functools.partial(_fe_kernel, G=G, Ho=Hh, Wo=Wh, Cin=C, N=N),
        out_shape=jax.ShapeDtypeStruct((B, N), jnp.float32),
        grid=(B // G,),
        in_specs=[pl.BlockSpec((G, C, S), lambda b: (b, 0, 0))]
        + [resident] * 14,
        out_specs=pl.BlockSpec((G, N), lambda b: (b, 0)),
        scratch_shapes=[
            pltpu.VMEM((4, Mt, C), jnp.bfloat16),
            pltpu.VMEM((Mt, 9 * N), jnp.bfloat16),
            pltpu.VMEM((Mt, N), jnp.bfloat16),
            pltpu.VMEM((Mt, N), jnp.bfloat16),
            pltpu.VMEM((Mt, N), jnp.float32),
        ],
        compiler_params=pltpu.CompilerParams(
            dimension_semantics=("parallel",),
            vmem_limit_bytes=56 * 1024 * 1024),
        cost_estimate=pl.CostEstimate(flops=flops, transcendentals=4 * B,
                                      bytes_accessed=bytes_accessed),
    )(x3, sel2, pmat,
      params["w11"], params["b11"], params["wds"], params["bds"],
      params["w12"], params["b12"], params["w21"], params["b21"],
      params["w22"], params["b22"], params["wfc"], params["bfc"])
    return out


def kernel(x, w11, b11, wds, bds, w12, b12, w21, b21, w22, b22, wfc, bfc):
    params = {"w11": w11, "b11": b11, "wds": wds, "bds": bds,
              "w12": w12, "b12": b12, "w21": w21, "b21": b21,
              "w22": w22, "b22": b22, "wfc": wfc, "bfc": bfc}
    return _fe_forward(x, params)


# speedup vs baseline: 1.0253x; 1.0096x over previous
import functools

import jax
import jax.numpy as jnp
from jax.experimental import pallas as pl
from jax.experimental.pallas import tpu as pltpu


_TAP2 = {-1: (1, -1), 0: (0, 0), 1: (1, 0)}

_RS = 8
_IS = 56


def _shift_to_patch(patch, read, s, c0, c1, mask=None):
    Mt = patch.shape[0]
    if s > 0:
        v = read(s, Mt)
        if mask is not None:
            v = jnp.where(mask[0:Mt - s], v, jnp.zeros_like(v))
            patch[Mt - s:Mt, c0:c1] = jnp.zeros((s, c1 - c0), patch.dtype)
        patch[0:Mt - s, c0:c1] = v
    elif s == 0:
        patch[:, c0:c1] = read(0, Mt)
    else:
        v = read(0, Mt + s)
        if mask is not None:
            v = jnp.where(mask[-s:Mt], v, jnp.zeros_like(v))
        patch[0:-s, c0:c1] = jnp.zeros((-s, c1 - c0), patch.dtype)
        patch[-s:Mt, c0:c1] = v


def _fe_kernel(x3_ref, sel_ref,
               w11_ref, b11_ref, wds_ref, bds_ref, w12_ref, b12_ref,
               w21_ref, b21_ref, w22_ref, b22_ref, wfc_ref, bfc_ref,
               o_ref,
               xp, patch, hA, y1f, idt_scr,
               *, G, Ho, Wo, Cin, N):
    Mt = G * _IS
    r = jax.lax.broadcasted_iota(jnp.int32, (Mt, 1), 0)
    rr = r % _IS
    valid = ((r % _RS) < Wo) & (rr < (Ho - 1) * _RS + Wo)
    i_of_r = rr // _RS
    m_top = i_of_r != 0
    m_bot = i_of_r != (Ho - 1)

    GB = 4 if G % 4 == 0 else 1
    for g in range(0, G, GB):
        xg = x3_ref[g:g + GB].reshape(GB * Cin, -1).astype(jnp.bfloat16)
        yg = jax.lax.dot_general(
            sel_ref[...], xg,
            (((1,), (1,)), ((), ())),
            preferred_element_type=jnp.float32).astype(jnp.bfloat16)
        for k in range(GB):
            for ph in range(4):
                xp[ph, (g + k) * _IS:(g + k + 1) * _IS, :] = \
                    yg[ph * _IS:(ph + 1) * _IS, k * Cin:(k + 1) * Cin]

    for ky in range(3):
        p, dy = _TAP2[ky - 1]
        for kx in range(3):
            q, dx = _TAP2[kx - 1]
            t = ky * 3 + kx
            _shift_to_patch(patch, lambda a, b: xp[2 * p + q, a:b, :],
                            dy * _RS + dx, t * Cin, (t + 1) * Cin,
                            mask=m_top if dy != 0 else None)
    acc = jnp.dot(patch[:, :9 * Cin], w11_ref[...],
                  preferred_element_type=jnp.float32) + b11_ref[...]
    hA[...] = jnp.where(valid, jnp.maximum(acc, 0.0), 0.0).astype(hA.dtype)

    idt_scr[...] = jnp.dot(xp[0], wds_ref[...],
                           preferred_element_type=jnp.float32) + bds_ref[...]

    for ky in range(3):
        for kx in range(3):
            t = ky * 3 + kx
            _shift_to_patch(patch, lambda a, b: hA[a:b, :],
                            (ky - 1) * _RS + (kx - 1), t * N, (t + 1) * N,
                            mask=(m_top, None, m_bot)[ky])
    acc = jnp.dot(patch[...], w12_ref[...],
                  preferred_element_type=jnp.float32) + b12_ref[...]
    y1f[...] = jnp.where(valid, jnp.maximum(acc + idt_scr[...], 0.0),
                         0.0).astype(y1f.dtype)

    for ky in range(3):
        for kx in range(3):
            t = ky * 3 + kx
            _shift_to_patch(patch, lambda a, b: y1f[a:b, :],
                            (ky - 1) * _RS + (kx - 1), t * N, (t + 1) * N,
                            mask=(m_top, None, m_bot)[ky])
    acc = jnp.dot(patch[...], w21_ref[...],
                  preferred_element_type=jnp.float32) + b21_ref[...]
    hA[...] = jnp.where(valid, jnp.maximum(acc, 0.0), 0.0).astype(hA.dtype)

    for ky in range(3):
        for kx in range(3):
            t = ky * 3 + kx
            _shift_to_patch(patch, lambda a, b: hA[a:b, :],
                            (ky - 1) * _RS + (kx - 1), t * N, (t + 1) * N,
                            mask=(m_top, None, m_bot)[ky])
    acc = jnp.dot(patch[...], w22_ref[...],
                  preferred_element_type=jnp.float32) + b22_ref[...]
    y2 = jnp.where(valid, jnp.maximum(acc + y1f[...].astype(jnp.float32), 0.0),
                   0.0)

    g_id = jax.lax.broadcasted_iota(jnp.int32, (G, Mt), 0)
    c_id = jax.lax.broadcasted_iota(jnp.int32, (G, Mt), 1)
    sel = ((c_id // _IS == g_id) & ((c_id % _RS) < Wo)
           & ((c_id % _IS) < (Ho - 1) * _RS + Wo))
    pmat = jnp.where(sel, jnp.float32(1.0 / (Ho * Wo)), jnp.float32(0.0))
    pooled = jnp.dot(pmat, y2, preferred_element_type=jnp.float32)

    h = jnp.dot(pooled, wfc_ref[...], preferred_element_type=jnp.float32)
    h = jnp.maximum(h + bfc_ref[...], 0.0)
    l2 = jnp.sqrt(jnp.sum(h * h, axis=1, keepdims=True))
    denom = jnp.sqrt(jnp.maximum(l2, 1e-12)) * jnp.float32(2.0 ** 0.5)
    o_ref[...] = h / denom


@jax.jit
def _fe_forward(x_nchw, params):
    B, C, H, W = x_nchw.shape
    Hh, Wh = H // 2, W // 2
    N = params["w11"].shape[1]
    G = 16 if B % 16 == 0 else B
    Mt = G * _IS
    S = H * W

    x3 = x_nchw.reshape(B, C, S)

    r_id = jax.lax.broadcasted_iota(jnp.int32, (4 * _IS, S), 0)
    s_id = jax.lax.broadcasted_iota(jnp.int32, (4 * _IS, S), 1)
    ph, rr = r_id // _IS, r_id % _IS
    i, j = rr // _RS, rr % _RS
    sigma = (2 * i + ph // 2) * W + 2 * j + ph % 2
    ok = (i < Hh) & (j < Wh)
    sel2 = ((s_id == sigma) & ok).astype(jnp.bfloat16)

    flops = B * (2 * Hh * Wh * 9 * C * N + 2 * Hh * Wh * C * N
                 + 3 * 2 * Hh * Wh * 9 * N * N + 2 * N * N)
    weight_keys = ("w11", "b11", "wds", "bds", "w12", "b12", "w21", "b21",
                   "w22", "b22", "wfc", "bfc")
    w_bytes = sum(int(params[k].size) * params[k].dtype.itemsize
                  for k in weight_keys)
    bytes_accessed = int(x3.size) * 4 + w_bytes + B * N * 4

    resident = pl.BlockSpec(memory_space=pltpu.MemorySpace.VMEM)

    out = pl.pallas_call(
        functools.partial(_fe_kernel, G=G, Ho=Hh, Wo=Wh, Cin=C, N=N),
        out_shape=jax.ShapeDtypeStruct((B, N), jnp.float32),
        grid=(B // G,),
        in_specs=[pl.BlockSpec((G, C, S), lambda b: (b, 0, 0))]
        + [resident] * 13,
        out_specs=pl.BlockSpec((G, N), lambda b: (b, 0)),
        scratch_shapes=[
            pltpu.VMEM((4, Mt, C), jnp.bfloat16),
            pltpu.VMEM((Mt, 9 * N), jnp.bfloat16),
            pltpu.VMEM((Mt, N), jnp.bfloat16),
            pltpu.VMEM((Mt, N), jnp.bfloat16),
            pltpu.VMEM((Mt, N), jnp.float32),
        ],
        compiler_params=pltpu.CompilerParams(
            dimension_semantics=("parallel",),
            vmem_limit_bytes=56 * 1024 * 1024),
        cost_estimate=pl.CostEstimate(flops=flops, transcendentals=4 * B,
                                      bytes_accessed=bytes_accessed),
    )(x3, sel2,
      params["w11"], params["b11"], params["wds"], params["bds"],
      params["w12"], params["b12"], params["w21"], params["b21"],
      params["w22"], params["b22"], params["wfc"], params["bfc"])
    return out


def kernel(x, w11, b11, wds, bds, w12, b12, w21, b21, w22, b22, wfc, bfc):
    params = {"w11": w11, "b11": b11, "wds": wds, "bds": bds,
              "w12": w12, "b12": b12, "w21": w21, "b21": b21,
              "w22": w22, "b22": b22, "wfc": wfc, "bfc": bfc}
    return _fe_forward(x, params)
